# baseline (speedup 1.0000x reference)
import jax
import jax.numpy as jnp
from jax import lax
from jax.experimental import pallas as pl
from jax.experimental.pallas import tpu as pltpu
from jax.experimental.pallas import tpu_sc as plsc

N = 10000
E = 320000
D = 128
K = 10
ALPHA = 0.1

L = 16
NS = 16
NC = 2
NP = 10240
CHUNK = 128
IG = 8
CT = 80
NG = CT // IG
EP = CHUNK * CT * NS * NC
RT = NP // NS
RC = RT // CHUNK


def _mlp_block(x_ref, w1_ref, b1_ref, w2_ref, b2_ref, o_ref):
    x = x_ref[...]
    h = lax.dot_general(x, w1_ref[...], (((1,), (1,)), ((), ())),
                        preferred_element_type=jnp.float32)
    h = jnp.maximum(h + b1_ref[...], 0.0)
    o = lax.dot_general(h, w2_ref[...], (((1,), (1,)), ((), ())),
                        preferred_element_type=jnp.float32)
    o_ref[...] = o + b2_ref[...]


def _mlp(xp, W1, b1, W2, b2):
    BR = 512
    return pl.pallas_call(
        _mlp_block,
        grid=(NP // BR,),
        in_specs=[
            pl.BlockSpec((BR, D), lambda i: (i, 0)),
            pl.BlockSpec((D, D), lambda i: (0, 0)),
            pl.BlockSpec((1, D), lambda i: (0, 0)),
            pl.BlockSpec((D, D), lambda i: (0, 0)),
            pl.BlockSpec((1, D), lambda i: (0, 0)),
        ],
        out_specs=pl.BlockSpec((BR, D), lambda i: (i, 0)),
        out_shape=jax.ShapeDtypeStruct((NP, D), jnp.float32),
    )(xp, W1, b1.reshape(1, D), W2, b2.reshape(1, D))


def _deg_body(idx_hbm, degp_hbm, deg_sp, i0, ones_t, zb):
    cid = lax.axis_index("c")
    sid = lax.axis_index("s")
    wid = cid * NS + sid
    base_g = wid * NG
    base_r = sid * RT

    zeros16 = jnp.zeros((L,), jnp.float32)
    ones16 = jnp.ones((L,), jnp.float32)

    def _z(i, c):
        zb[pl.ds(i * L, L)] = zeros16
        return c
    lax.fori_loop(0, RT // L, _z, 0)

    def _o(i, c):
        ones_t[pl.ds(i * L, L)] = ones16
        return c
    lax.fori_loop(0, CHUNK // L, _o, 0)

    pltpu.sync_copy(zb, deg_sp.at[pl.ds(base_r, RT)])
    plsc.subcore_barrier()

    def _degg(g, carry):
        pltpu.sync_copy(idx_hbm.at[base_g + g], i0)

        def _deg(j, c2):
            pltpu.sync_copy(ones_t, deg_sp.at[i0.at[IG + j]], add=True)
            return c2
        lax.fori_loop(0, IG, _deg, 0)
        return carry
    lax.fori_loop(0, NG, _degg, 0)
    plsc.subcore_barrier()

    pltpu.sync_copy(deg_sp.at[pl.ds(base_r, RT)],
                    degp_hbm.at[cid, pl.ds(base_r, RT)])


def _degrees(idx_p):
    mesh = plsc.VectorSubcoreMesh(core_axis_name="c", subcore_axis_name="s",
                                  num_cores=NC, num_subcores=NS)
    fn = pl.kernel(
        _deg_body,
        jax.ShapeDtypeStruct((NC, NP), jnp.float32),
        mesh=mesh,
        scratch_types=[
            pltpu.VMEM_SHARED((NP,), jnp.float32),
            pltpu.VMEM((2 * IG, CHUNK), jnp.int32),
            pltpu.VMEM((CHUNK,), jnp.float32),
            pltpu.VMEM((RT,), jnp.float32),
        ],
    )
    return fn(idx_p)


def _edge_body(idx_hbm, y_hbm, accp_hbm,
               acc_sp, i0, i1, g0, g1,
               isem0, isem1, gsem0, gsem1):
    cid = lax.axis_index("c")
    sid = lax.axis_index("s")
    wid = cid * NS + sid
    base_g = wid * NG
    base_r = sid * RT

    zeros16 = jnp.zeros((L,), jnp.float32)

    def _zr(r, c):
        for cc in range(D // L):
            g0[r, pl.ds(cc * L, L)] = zeros16
        return c
    lax.fori_loop(0, CHUNK, _zr, 0)

    def _za(b, c):
        pltpu.sync_copy(g0, acc_sp.at[pl.ds(base_r + b * CHUNK, CHUNK), :])
        return c
    lax.fori_loop(0, RC, _za, 0)
    plsc.subcore_barrier()

    gbufs = (g0, g1)
    gsems = (gsem0, gsem1)
    ibufs = (i0, i1)
    isems = (isem0, isem1)

    @pl.when(cid == 0)
    def _():
        def _half(hb, c0):
            bg = hb * NS * NG + sid * NG
            pltpu.async_copy(idx_hbm.at[bg], i0, isem0)
            pltpu.async_copy(idx_hbm.at[bg + 1], i1, isem1)

            def _gpair(p, c1):
                for b in range(2):
                    g = 2 * p + b
                    ib = ibufs[b]
                    pltpu.make_async_copy(idx_hbm.at[bg + g], ib,
                                          isems[b]).wait()
                    pltpu.async_copy(y_hbm.at[ib.at[0]], gbufs[0], gsems[0])
                    pltpu.async_copy(y_hbm.at[ib.at[1]], gbufs[1], gsems[1])
                    for c in range(IG):
                        bb = c % 2
                        pltpu.make_async_copy(y_hbm.at[ib.at[c]], gbufs[bb],
                                              gsems[bb]).wait()
                        pltpu.sync_copy(gbufs[bb], acc_sp.at[ib.at[IG + c]],
                                        add=True)
                        if c + 2 < IG:
                            pltpu.async_copy(y_hbm.at[ib.at[c + 2]],
                                             gbufs[bb], gsems[bb])

                    @pl.when(g + 2 < NG)
                    def _():
                        pltpu.async_copy(idx_hbm.at[bg + g + 2], ib,
                                         isems[b])
                return c1
            lax.fori_loop(0, NG // 2, _gpair, 0)
            return c0
        lax.fori_loop(0, NC, _half, 0)
    plsc.subcore_barrier()

    pltpu.sync_copy(acc_sp.at[pl.ds(base_r, RT), :],
                    accp_hbm.at[cid, pl.ds(base_r, RT), :])


def _edge_pass(idx_p, y):
    mesh = plsc.VectorSubcoreMesh(core_axis_name="c", subcore_axis_name="s",
                                  num_cores=NC, num_subcores=NS)
    fn = pl.kernel(
        _edge_body,
        jax.ShapeDtypeStruct((NC, NP, D), jnp.float32),
        mesh=mesh,
        scratch_types=[
            pltpu.VMEM_SHARED((NP, D), jnp.float32),
            pltpu.VMEM((2 * IG, CHUNK), jnp.int32),
            pltpu.VMEM((2 * IG, CHUNK), jnp.int32),
            pltpu.VMEM((CHUNK, D), jnp.float32),
            pltpu.VMEM((CHUNK, D), jnp.float32),
            pltpu.SemaphoreType.DMA,
            pltpu.SemaphoreType.DMA,
            pltpu.SemaphoreType.DMA,
            pltpu.SemaphoreType.DMA,
        ],
    )
    return fn(idx_p, y)


def _seed_block(degp_ref, h_ref, dinv_ref, y_ref):
    deg = degp_ref[0] + degp_ref[1] + 1.0
    dinv = lax.rsqrt(deg)
    dinv_ref[...] = dinv
    y_ref[...] = dinv[:, None] * h_ref[...]


def _seed(degp, h):
    BR = 1024
    return pl.pallas_call(
        _seed_block,
        grid=(NP // BR,),
        in_specs=[
            pl.BlockSpec((NC, BR), lambda i: (0, i)),
            pl.BlockSpec((BR, D), lambda i: (i, 0)),
        ],
        out_specs=[
            pl.BlockSpec((BR,), lambda i: (i,)),
            pl.BlockSpec((BR, D), lambda i: (i, 0)),
        ],
        out_shape=[
            jax.ShapeDtypeStruct((NP,), jnp.float32),
            jax.ShapeDtypeStruct((NP, D), jnp.float32),
        ],
    )(degp, h)


def _combine_block(accp_ref, y_ref, h_ref, dinv_ref, x_ref, yn_ref):
    dinv = dinv_ref[...][:, None]
    s = y_ref[...] + accp_ref[0] + accp_ref[1]
    xv = (1.0 - ALPHA) * dinv * s + ALPHA * h_ref[...]
    x_ref[...] = xv
    yn_ref[...] = dinv * xv


def _combine(accp, y, h, dinv):
    BR = 1024
    return pl.pallas_call(
        _combine_block,
        grid=(NP // BR,),
        in_specs=[
            pl.BlockSpec((NC, BR, D), lambda i: (0, i, 0)),
            pl.BlockSpec((BR, D), lambda i: (i, 0)),
            pl.BlockSpec((BR, D), lambda i: (i, 0)),
            pl.BlockSpec((BR,), lambda i: (i,)),
        ],
        out_specs=[
            pl.BlockSpec((BR, D), lambda i: (i, 0)),
            pl.BlockSpec((BR, D), lambda i: (i, 0)),
        ],
        out_shape=[
            jax.ShapeDtypeStruct((NP, D), jnp.float32),
            jax.ShapeDtypeStruct((NP, D), jnp.float32),
        ],
    )(accp, y, h, dinv)


def kernel(x, edge_index, W1, b1, W2, b2):
    xp = jnp.pad(x, ((0, NP - N), (0, 0)))
    h = _mlp(xp, W1, b1, W2, b2)

    rows = edge_index[0]
    cols = edge_index[1]
    pad = EP - E
    nw = NC * NS
    rows_p = jnp.concatenate(
        [rows, jnp.zeros((pad,), jnp.int32)]).reshape(nw * NG, IG, CHUNK)
    cols_p = jnp.concatenate(
        [cols, jnp.full((pad,), NP - 1, jnp.int32)]).reshape(nw * NG, IG, CHUNK)
    idx_p = jnp.concatenate([rows_p, cols_p], axis=1)

    degp = _degrees(idx_p)
    dinv, y = _seed(degp, h)

    xk = h
    for _ in range(K):
        accp = _edge_pass(idx_p, y)
        xk, y = _combine(accp, y, h, dinv)
    return xk[:N]


# speedup vs baseline: 1.4106x; 1.4106x over previous
import jax
import jax.numpy as jnp
from jax import lax
from jax.experimental import pallas as pl
from jax.experimental.pallas import tpu as pltpu
from jax.experimental.pallas import tpu_sc as plsc

N = 10000
E = 320000
D = 128
K = 10
ALPHA = 0.1

L = 16
NS = 16
NC = 2
NP = 10240
NH = NP // 2
CHUNK = 128
IG = 8
CT = 80
NG = CT // IG
EP = CHUNK * CT * NS * NC
RT = NP // NS
OT = NH // NS
CB = 64
ZR = 32


def _mlp_block(x_ref, w1_ref, b1_ref, w2_ref, b2_ref, o_ref):
    x = x_ref[...]
    h = lax.dot_general(x, w1_ref[...], (((1,), (1,)), ((), ())),
                        preferred_element_type=jnp.float32)
    h = jnp.maximum(h + b1_ref[...], 0.0)
    o = lax.dot_general(h, w2_ref[...], (((1,), (1,)), ((), ())),
                        preferred_element_type=jnp.float32)
    o_ref[...] = o + b2_ref[...]


def _mlp(xp, W1, b1, W2, b2):
    BR = 512
    return pl.pallas_call(
        _mlp_block,
        grid=(NP // BR,),
        in_specs=[
            pl.BlockSpec((BR, D), lambda i: (i, 0)),
            pl.BlockSpec((D, D), lambda i: (0, 0)),
            pl.BlockSpec((1, D), lambda i: (0, 0)),
            pl.BlockSpec((D, D), lambda i: (0, 0)),
            pl.BlockSpec((1, D), lambda i: (0, 0)),
        ],
        out_specs=pl.BlockSpec((BR, D), lambda i: (i, 0)),
        out_shape=jax.ShapeDtypeStruct((NP, D), jnp.float32),
    )(xp, W1, b1.reshape(1, D), W2, b2.reshape(1, D))


def _prop_body(idx_hbm, h_hbm, x_out, y_hbm, dinv_hbm, xb_hbm,
               acc_sp, deg_sp,
               i0, i1, g0, g1, zb2, dinv_t, ones_t, zb,
               gsem0, gsem1, isem0, isem1, xsem):
    cid = lax.axis_index("c")
    sid = lax.axis_index("s")
    ocid = 1 - cid

    def _xbarrier():
        plsc.subcore_barrier()

        @pl.when(sid == 0)
        def _():
            pl.semaphore_signal(xsem, 1, core_index=ocid)
            pl.semaphore_wait(xsem, 1)
        plsc.subcore_barrier()

    zeros16 = jnp.zeros((L,), jnp.float32)

    def _z2(i, c):
        r = i // (D // L)
        cc = i % (D // L)
        zb2[r, pl.ds(cc * L, L)] = zeros16
        return c
    lax.fori_loop(0, ZR * (D // L), _z2, 0)

    @pl.when(cid == 0)
    def _():
        base_r = sid * RT
        ones16 = jnp.ones((L,), jnp.float32)

        def _z(i, c):
            zb[pl.ds(i * L, L)] = zeros16
            return c
        lax.fori_loop(0, RT // L, _z, 0)

        def _o(i, c):
            ones_t[pl.ds(i * L, L)] = ones16
            return c
        lax.fori_loop(0, CHUNK // L, _o, 0)

        pltpu.sync_copy(zb, deg_sp.at[pl.ds(base_r, RT)])
        plsc.subcore_barrier()

        def _degh(hb, c0):
            bg = (hb * NS + sid) * NG

            def _degg(g, carry):
                pltpu.sync_copy(idx_hbm.at[bg + g], i0)

                def _deg(j, c2):
                    pltpu.sync_copy(ones_t, deg_sp.at[i0.at[IG + j]],
                                    add=True)
                    return c2
                lax.fori_loop(0, IG, _deg, 0)
                return carry
            lax.fori_loop(0, NG, _degg, 0)
            return c0
        lax.fori_loop(0, NC, _degh, 0)
        plsc.subcore_barrier()

        pltpu.sync_copy(deg_sp.at[pl.ds(base_r, RT)], zb)

        def _rs(i, carry):
            d = zb[pl.ds(i * L, L)] + 1.0
            s = 0.5 * (d + 1.0)
            for _ in range(15):
                s = 0.5 * (s + d / s)
            zb[pl.ds(i * L, L)] = 1.0 / s
            return carry
        lax.fori_loop(0, RT // L, _rs, 0)
        pltpu.sync_copy(zb, dinv_hbm.at[pl.ds(base_r, RT)])

        def _init(cj, carry):
            r0 = base_r + cj * CHUNK
            pltpu.sync_copy(h_hbm.at[pl.ds(r0, CHUNK), :], g0)

            def _row(rg, c2):
                dv = zb[pl.ds(cj * CHUNK + rg * L, L)]
                for j in range(L):
                    s = dv[j]
                    r = rg * L + j
                    for cc in range(D // L):
                        v = g0[r, pl.ds(cc * L, L)]
                        g0[r, pl.ds(cc * L, L)] = s * v
                return c2
            lax.fori_loop(0, CHUNK // L, _row, 0)

            pltpu.sync_copy(g0, y_hbm.at[pl.ds(r0, CHUNK), :])

            @pl.when(sid < NS // 2)
            def _():
                pltpu.sync_copy(g0, acc_sp.at[pl.ds(r0, CHUNK), :])
            return carry
        lax.fori_loop(0, RT // CHUNK, _init, 0)

        @pl.when(sid >= NS // 2)
        def _():
            def _za(b, c):
                pltpu.sync_copy(zb2, acc_sp.at[pl.ds(base_r + b * ZR, ZR), :])
                return c
            lax.fori_loop(0, RT // ZR, _za, 0)

        plsc.subcore_barrier()

        @pl.when(sid == 0)
        def _():
            pl.semaphore_signal(xsem, 1, core_index=1)

    @pl.when(cid == 1)
    def _():
        base_r = sid * RT

        @pl.when(sid == 0)
        def _():
            pl.semaphore_wait(xsem, 1)
        plsc.subcore_barrier()

        def _za(b, c):
            pltpu.sync_copy(zb2, acc_sp.at[pl.ds(base_r + b * ZR, ZR), :])
            return c

        @pl.when(sid < NS // 2)
        def _():
            lax.fori_loop(0, RT // ZR, _za, 0)

        @pl.when(sid >= NS // 2)
        def _():
            def _cp(b, c):
                r0 = base_r + b * CHUNK
                pltpu.sync_copy(y_hbm.at[pl.ds(r0, CHUNK), :], g0)
                pltpu.sync_copy(g0, acc_sp.at[pl.ds(r0, CHUNK), :])
                return c
            lax.fori_loop(0, RT // CHUNK, _cp, 0)

    own0 = cid * NH + sid * OT
    pltpu.sync_copy(dinv_hbm.at[pl.ds(own0, OT)], dinv_t)

    gbufs = (g0, g1)
    gsems = (gsem0, gsem1)
    ibufs = (i0, i1)
    isems = (isem0, isem1)
    base_g = (cid * NS + sid) * NG
    oth0 = ocid * NH + sid * OT
    xslice = sid * OT

    def _step(k, carry):
        plsc.subcore_barrier()

        pltpu.async_copy(idx_hbm.at[base_g], i0, isem0)
        pltpu.async_copy(idx_hbm.at[base_g + 1], i1, isem1)

        def _gpair(p, c1):
            for b in range(2):
                g = 2 * p + b
                ib = ibufs[b]
                pltpu.make_async_copy(idx_hbm.at[base_g + g], ib,
                                      isems[b]).wait()
                pltpu.async_copy(y_hbm.at[ib.at[0]], gbufs[0], gsems[0])
                pltpu.async_copy(y_hbm.at[ib.at[1]], gbufs[1], gsems[1])
                for c in range(IG):
                    bb = c % 2
                    pltpu.make_async_copy(y_hbm.at[ib.at[c]], gbufs[bb],
                                          gsems[bb]).wait()
                    pltpu.sync_copy(gbufs[bb], acc_sp.at[ib.at[IG + c]],
                                    add=True)
                    if c + 2 < IG:
                        pltpu.async_copy(y_hbm.at[ib.at[c + 2]],
                                         gbufs[bb], gsems[bb])

                @pl.when(g + 2 < NG)
                def _():
                    pltpu.async_copy(idx_hbm.at[base_g + g + 2], ib,
                                     isems[b])
            return c1
        lax.fori_loop(0, NG // 2, _gpair, 0)
        plsc.subcore_barrier()

        pltpu.sync_copy(acc_sp.at[pl.ds(oth0, OT), :],
                        xb_hbm.at[cid, pl.ds(xslice, OT), :])

        def _zo(b, c):
            pltpu.sync_copy(zb2, acc_sp.at[pl.ds(oth0 + b * ZR, ZR), :])
            return c
        lax.fori_loop(0, OT // ZR, _zo, 0)

        _xbarrier()

        def _comb(cj, c3):
            r0 = own0 + cj * CB
            rx = xslice + cj * CB
            pltpu.sync_copy(acc_sp.at[pl.ds(r0, CB), :], g0.at[pl.ds(0, CB)])
            pltpu.sync_copy(xb_hbm.at[ocid, pl.ds(rx, CB), :],
                            g0.at[pl.ds(CB, CB)])
            pltpu.sync_copy(h_hbm.at[pl.ds(r0, CB), :], g1.at[pl.ds(0, CB)])

            def _row(rg, c4):
                dv = dinv_t[pl.ds(cj * CB + rg * L, L)]
                for j in range(L):
                    s = dv[j]
                    a = (1.0 - ALPHA) * s
                    r = rg * L + j
                    for cc in range(D // L):
                        va = g0[r, pl.ds(cc * L, L)]
                        vb = g0[CB + r, pl.ds(cc * L, L)]
                        vh = g1[r, pl.ds(cc * L, L)]
                        xv = a * (va + vb) + ALPHA * vh
                        g1[r, pl.ds(cc * L, L)] = xv
                        g0[r, pl.ds(cc * L, L)] = s * xv
                return c4
            lax.fori_loop(0, CB // L, _row, 0)

            pltpu.sync_copy(g1.at[pl.ds(0, CB)], x_out.at[pl.ds(r0, CB), :])
            pltpu.sync_copy(g0.at[pl.ds(0, CB)], y_hbm.at[pl.ds(r0, CB), :])
            pltpu.sync_copy(g0.at[pl.ds(0, CB)], acc_sp.at[pl.ds(r0, CB), :])
            return c3
        lax.fori_loop(0, OT // CB, _comb, 0)

        _xbarrier()
        return carry
    lax.fori_loop(0, K, _step, 0)


def _propagate(idx_p, h):
    mesh = plsc.VectorSubcoreMesh(core_axis_name="c", subcore_axis_name="s",
                                  num_cores=NC, num_subcores=NS)
    out_type = (jax.ShapeDtypeStruct((NP, D), jnp.float32),
                jax.ShapeDtypeStruct((NP, D), jnp.float32),
                jax.ShapeDtypeStruct((NP,), jnp.float32),
                jax.ShapeDtypeStruct((NC, NH, D), jnp.float32))
    fn = pl.kernel(
        _prop_body,
        out_type,
        mesh=mesh,
        scratch_types=[
            pltpu.VMEM_SHARED((NP, D), jnp.float32),
            pltpu.VMEM_SHARED((NP,), jnp.float32),
            pltpu.VMEM((2 * IG, CHUNK), jnp.int32),
            pltpu.VMEM((2 * IG, CHUNK), jnp.int32),
            pltpu.VMEM((CHUNK, D), jnp.float32),
            pltpu.VMEM((CHUNK, D), jnp.float32),
            pltpu.VMEM((ZR, D), jnp.float32),
            pltpu.VMEM((OT,), jnp.float32),
            pltpu.VMEM((CHUNK,), jnp.float32),
            pltpu.VMEM((RT,), jnp.float32),
            pltpu.SemaphoreType.DMA,
            pltpu.SemaphoreType.DMA,
            pltpu.SemaphoreType.DMA,
            pltpu.SemaphoreType.DMA,
            pltpu.SemaphoreType.REGULAR,
        ],
    )
    return fn(idx_p, h)


def kernel(x, edge_index, W1, b1, W2, b2):
    xp = jnp.pad(x, ((0, NP - N), (0, 0)))
    h = _mlp(xp, W1, b1, W2, b2)

    rows = edge_index[0]
    cols = edge_index[1]
    pad = EP - E
    nw = NC * NS
    rows_p = jnp.concatenate(
        [rows, jnp.zeros((pad,), jnp.int32)]).reshape(nw * NG, IG, CHUNK)
    cols_p = jnp.concatenate(
        [cols, jnp.full((pad,), NP - 1, jnp.int32)]).reshape(nw * NG, IG, CHUNK)
    idx_p = jnp.concatenate([rows_p, cols_p], axis=1)

    xk, _, _, _ = _propagate(idx_p, h)
    return xk[:N]
